# baseline (speedup 1.0000x reference)
import functools
import math

import jax
import jax.numpy as jnp
from jax import lax
from jax.experimental import pallas as pl
from jax.experimental.pallas import tpu as pltpu
from jax.experimental.pallas import tpu_sc as plsc

N = 10000
E = 320000
D = 128
NC = 2
NS = 16
NW = NC * NS
EPT = E // NW
CHUNK = 80
NCHUNK = 128
EPAD = NCHUNK * CHUNK - EPT
NBUF = 1
NACC = N + NS
RPS = 624
TAIL = N - NS * RPS

_INV = 1.0 / math.sqrt(1.0 + 1e-5)


def _segment_sum_sc(h, src3, dst3, zeros):
    mesh = plsc.VectorSubcoreMesh(
        core_axis_name="c", subcore_axis_name="s", num_cores=NC, num_subcores=NS
    )

    @functools.partial(
        pl.kernel,
        out_type=jax.ShapeDtypeStruct((NC, N, D), jnp.float32),
        mesh=mesh,
        scratch_types=[
            pltpu.VMEM((CHUNK,), jnp.int32),
            pltpu.VMEM((CHUNK,), jnp.int32),
            [pltpu.VMEM((CHUNK, D), jnp.float32) for _ in range(NBUF)],
            pltpu.VMEM_SHARED((NACC, D), jnp.float32),
            [pltpu.SemaphoreType.DMA for _ in range(NBUF)],
            [pltpu.SemaphoreType.DMA for _ in range(NBUF)],
        ],
    )
    def k(h_hbm, src_hbm, dst_hbm, z_hbm, out_hbm, srcv, dstv, rows, acc,
          gsem, ssem):
        cid = lax.axis_index("c")
        sid = lax.axis_index("s")
        wid = sid * NC + cid
        r0 = sid * RPS

        pltpu.sync_copy(z_hbm.at[pl.ds(r0, RPS)], acc.at[pl.ds(r0, RPS)])

        @pl.when(sid == NS - 1)
        def _():
            pltpu.sync_copy(z_hbm.at[pl.ds(NS * RPS, TAIL)],
                            acc.at[pl.ds(NS * RPS, TAIL)])

        plsc.subcore_barrier()

        base = wid * NCHUNK * CHUNK

        @pl.loop(0, NCHUNK)
        def _(i):
            off = base + i * CHUNK
            pltpu.sync_copy(src_hbm.at[pl.ds(off, CHUNK)], srcv)
            pltpu.sync_copy(dst_hbm.at[pl.ds(off, CHUNK)], dstv)
            pltpu.async_copy(h_hbm.at[srcv], rows[0], gsem[0]).wait()
            pltpu.sync_copy(rows[0], acc.at[dstv], add=True)

        plsc.subcore_barrier()
        pltpu.sync_copy(acc.at[pl.ds(r0, RPS)], out_hbm.at[cid].at[pl.ds(r0, RPS)])

        @pl.when(sid == NS - 1)
        def _():
            pltpu.sync_copy(acc.at[pl.ds(NS * RPS, TAIL)],
                            out_hbm.at[cid].at[pl.ds(NS * RPS, TAIL)])

    return k(h, src3, dst3, zeros)


def _mlp0_body(x_ref, p0_ref, p1_ref, w1t_ref, b1_ref, g1_ref, be1_ref,
               w2t_ref, b2_ref, g0_ref, be0_ref, o_ref):
    t = x_ref[...] + p0_ref[...] + p1_ref[...]
    a = jnp.dot(t, w1t_ref[...], preferred_element_type=jnp.float32) + b1_ref[...]
    a = jnp.maximum(a * (_INV * g1_ref[...]) + be1_ref[...], 0.0)
    h = jnp.dot(a, w2t_ref[...], preferred_element_type=jnp.float32) + b2_ref[...]
    o_ref[...] = jnp.maximum(h * (_INV * g0_ref[...]) + be0_ref[...], 0.0)


def _head_body(h_ref, p0_ref, p1_ref, w1t_ref, b1_ref, g1_ref, be1_ref,
               w2t_ref, b2_ref, wct_ref, bc_ref, wst_ref, bs_ref,
               wmt_ref, bm_ref, main_ref, sim_ref, he_ref):
    t = h_ref[...] + p0_ref[...] + p1_ref[...]
    a = jnp.dot(t, w1t_ref[...], preferred_element_type=jnp.float32) + b1_ref[...]
    a = jnp.maximum(a * (_INV * g1_ref[...]) + be1_ref[...], 0.0)
    h2 = jnp.dot(a, w2t_ref[...], preferred_element_type=jnp.float32) + b2_ref[...]

    main = jnp.dot(h2, wct_ref[...], preferred_element_type=jnp.float32) + bc_ref[...]
    m = jnp.max(main, axis=-1, keepdims=True)
    s = main - m
    main_ref[...] = s - jnp.log(jnp.sum(jnp.exp(s), axis=-1, keepdims=True))

    sim = jnp.dot(h2, wst_ref[...], preferred_element_type=jnp.float32) + bs_ref[...]
    ms = jnp.max(sim, axis=-1, keepdims=True)
    es = jnp.exp(sim - ms)
    sim_ref[...] = es / jnp.sum(es, axis=-1, keepdims=True)

    he = jnp.dot(h2, wmt_ref[...], preferred_element_type=jnp.float32) + bm_ref[...]
    he_ref[...] = 1.0 / (1.0 + jnp.exp(-he))


_BM = 1000


def _row(i):
    return (i, 0)


def _fixed(i):
    return (0, 0)


def _mlp0(x, p0, p1, w1t, b1, g1, be1, w2t, b2, g0, be0):
    rspec = pl.BlockSpec((_BM, D), _row)
    wspec = pl.BlockSpec((D, D), _fixed)
    vspec = pl.BlockSpec((1, D), _fixed)
    return pl.pallas_call(
        _mlp0_body,
        out_shape=jax.ShapeDtypeStruct((N, D), jnp.float32),
        grid=(N // _BM,),
        in_specs=[rspec, rspec, rspec, wspec, vspec, vspec, vspec,
                  wspec, vspec, vspec, vspec],
        out_specs=rspec,
    )(x, p0, p1, w1t, b1, g1, be1, w2t, b2, g0, be0)


def _heads(h, p0, p1, w1t, b1, g1, be1, w2t, b2, wct, bc, wst, bs, wmt, bm):
    rspec = pl.BlockSpec((_BM, D), _row)
    wspec = pl.BlockSpec((D, D), _fixed)
    vspec = pl.BlockSpec((1, D), _fixed)
    return pl.pallas_call(
        _head_body,
        out_shape=(
            jax.ShapeDtypeStruct((N, 40), jnp.float32),
            jax.ShapeDtypeStruct((N, 40), jnp.float32),
            jax.ShapeDtypeStruct((N, 2), jnp.float32),
        ),
        grid=(N // _BM,),
        in_specs=[rspec, rspec, rspec, wspec, vspec, vspec, vspec,
                  wspec, vspec,
                  pl.BlockSpec((D, 40), _fixed), pl.BlockSpec((1, 40), _fixed),
                  pl.BlockSpec((D, 40), _fixed), pl.BlockSpec((1, 40), _fixed),
                  pl.BlockSpec((D, 2), _fixed), pl.BlockSpec((1, 2), _fixed)],
        out_specs=(
            pl.BlockSpec((_BM, 40), _row),
            pl.BlockSpec((_BM, 40), _row),
            pl.BlockSpec((_BM, 2), _row),
        ),
    )(h, p0, p1, w1t, b1, g1, be1, w2t, b2, wct, bc, wst, bs, wmt, bm)


def kernel(x, edge_index, params):
    src = edge_index[0].astype(jnp.int32)
    dst = edge_index[1].astype(jnp.int32)
    src3 = jnp.pad(src.reshape(NW, EPT), ((0, 0), (0, EPAD)),
                   constant_values=0).reshape(NW * NCHUNK * CHUNK)
    trash = (N + jnp.arange(NW, dtype=jnp.int32) // NC)[:, None]
    dst3 = jnp.concatenate(
        [dst.reshape(NW, EPT),
         jnp.broadcast_to(trash, (NW, EPAD))], axis=1).reshape(NW * NCHUNK * CHUNK)
    zeros = jnp.zeros((N, D), jnp.float32)

    c0, c1 = params["conv0"], params["conv1"]

    def vec(v):
        return v.reshape(1, -1)

    parts0 = _segment_sum_sc(x, src3, dst3, zeros)
    h1 = _mlp0(
        x, parts0[0], parts0[1],
        c0["lin1"]["W"].T, vec(c0["lin1"]["b"]), vec(c0["bn"]["g"]), vec(c0["bn"]["be"]),
        c0["lin2"]["W"].T, vec(c0["lin2"]["b"]),
        vec(params["bn0"]["g"]), vec(params["bn0"]["be"]),
    )

    parts1 = _segment_sum_sc(h1, src3, dst3, zeros)
    wmt = jnp.concatenate([params["homo"]["W"].T, params["ent"]["W"].T], axis=1)
    bm = jnp.concatenate([params["homo"]["b"], params["ent"]["b"]]).reshape(1, 2)
    main, sim, he = _heads(
        h1, parts1[0], parts1[1],
        c1["lin1"]["W"].T, vec(c1["lin1"]["b"]), vec(c1["bn"]["g"]), vec(c1["bn"]["be"]),
        c1["lin2"]["W"].T, vec(c1["lin2"]["b"]),
        params["cls"]["W"].T, vec(params["cls"]["b"]),
        params["sim"]["W"].T, vec(params["sim"]["b"]),
        wmt, bm,
    )
    return main, sim, he[:, 0], he[:, 1]


# speedup vs baseline: 1.7739x; 1.7739x over previous
import functools
import math

import jax
import jax.numpy as jnp
from jax import lax
from jax.experimental import pallas as pl
from jax.experimental.pallas import tpu as pltpu
from jax.experimental.pallas import tpu_sc as plsc

N = 10000
E = 320000
D = 128
NC = 2
NS = 16
NW = NC * NS
EPT = E // NW
CHUNK = 80
NCHUNK = 125
EPAD = NCHUNK * CHUNK - EPT
NBUF = 1
NACC = N
RPS = 624
TAIL = N - NS * RPS

_INV = 1.0 / math.sqrt(1.0 + 1e-5)


def _segment_sum_sc(h, src3, dst3, zeros):
    mesh = plsc.VectorSubcoreMesh(
        core_axis_name="c", subcore_axis_name="s", num_cores=NC, num_subcores=NS
    )

    @functools.partial(
        pl.kernel,
        out_type=jax.ShapeDtypeStruct((NC, N, D), jnp.float32),
        mesh=mesh,
        scratch_types=[
            pltpu.VMEM((CHUNK,), jnp.int32),
            pltpu.VMEM((CHUNK,), jnp.int32),
            [pltpu.VMEM((CHUNK, D), jnp.float32) for _ in range(NBUF)],
            pltpu.VMEM_SHARED((NACC, D), jnp.float32),
            [pltpu.SemaphoreType.DMA for _ in range(NBUF)],
            [pltpu.SemaphoreType.DMA for _ in range(NBUF)],
        ],
    )
    def k(h_hbm, src_hbm, dst_hbm, z_hbm, out_hbm, srcv, dstv, rows, acc,
          gsem, ssem):
        cid = lax.axis_index("c")
        sid = lax.axis_index("s")
        wid = sid * NC + cid
        r0 = sid * RPS

        pltpu.sync_copy(z_hbm.at[pl.ds(r0, RPS)], acc.at[pl.ds(r0, RPS)])

        @pl.when(sid == NS - 1)
        def _():
            pltpu.sync_copy(z_hbm.at[pl.ds(NS * RPS, TAIL)],
                            acc.at[pl.ds(NS * RPS, TAIL)])

        plsc.subcore_barrier()

        base = wid * NCHUNK * CHUNK

        @pl.loop(0, NCHUNK)
        def _(i):
            off = base + i * CHUNK
            pltpu.sync_copy(src_hbm.at[pl.ds(off, CHUNK)], srcv)
            pltpu.sync_copy(dst_hbm.at[pl.ds(off, CHUNK)], dstv)
            pltpu.async_copy(h_hbm.at[srcv], rows[0], gsem[0]).wait()
            pltpu.sync_copy(rows[0], acc.at[dstv], add=True)

        plsc.subcore_barrier()
        pltpu.sync_copy(acc.at[pl.ds(r0, RPS)], out_hbm.at[cid].at[pl.ds(r0, RPS)])

        @pl.when(sid == NS - 1)
        def _():
            pltpu.sync_copy(acc.at[pl.ds(NS * RPS, TAIL)],
                            out_hbm.at[cid].at[pl.ds(NS * RPS, TAIL)])

    return k(h, src3, dst3, zeros)


def _mlp0_body(x_ref, p0_ref, p1_ref, w1t_ref, b1_ref, g1_ref, be1_ref,
               w2t_ref, b2_ref, g0_ref, be0_ref, o_ref):
    t = x_ref[...] + p0_ref[...] + p1_ref[...]
    a = jnp.dot(t, w1t_ref[...], preferred_element_type=jnp.float32) + b1_ref[...]
    a = jnp.maximum(a * (_INV * g1_ref[...]) + be1_ref[...], 0.0)
    h = jnp.dot(a, w2t_ref[...], preferred_element_type=jnp.float32) + b2_ref[...]
    o_ref[...] = jnp.maximum(h * (_INV * g0_ref[...]) + be0_ref[...], 0.0)


def _head_body(h_ref, p0_ref, p1_ref, w1t_ref, b1_ref, g1_ref, be1_ref,
               w2t_ref, b2_ref, wct_ref, bc_ref, wst_ref, bs_ref,
               wmt_ref, bm_ref, main_ref, sim_ref, he_ref):
    t = h_ref[...] + p0_ref[...] + p1_ref[...]
    a = jnp.dot(t, w1t_ref[...], preferred_element_type=jnp.float32) + b1_ref[...]
    a = jnp.maximum(a * (_INV * g1_ref[...]) + be1_ref[...], 0.0)
    h2 = jnp.dot(a, w2t_ref[...], preferred_element_type=jnp.float32) + b2_ref[...]

    main = jnp.dot(h2, wct_ref[...], preferred_element_type=jnp.float32) + bc_ref[...]
    m = jnp.max(main, axis=-1, keepdims=True)
    s = main - m
    main_ref[...] = s - jnp.log(jnp.sum(jnp.exp(s), axis=-1, keepdims=True))

    sim = jnp.dot(h2, wst_ref[...], preferred_element_type=jnp.float32) + bs_ref[...]
    ms = jnp.max(sim, axis=-1, keepdims=True)
    es = jnp.exp(sim - ms)
    sim_ref[...] = es / jnp.sum(es, axis=-1, keepdims=True)

    he = jnp.dot(h2, wmt_ref[...], preferred_element_type=jnp.float32) + bm_ref[...]
    he_ref[...] = 1.0 / (1.0 + jnp.exp(-he))


_BM = 1000


def _row(i):
    return (i, 0)


def _fixed(i):
    return (0, 0)


def _mlp0(x, p0, p1, w1t, b1, g1, be1, w2t, b2, g0, be0):
    rspec = pl.BlockSpec((_BM, D), _row)
    wspec = pl.BlockSpec((D, D), _fixed)
    vspec = pl.BlockSpec((1, D), _fixed)
    return pl.pallas_call(
        _mlp0_body,
        out_shape=jax.ShapeDtypeStruct((N, D), jnp.float32),
        grid=(N // _BM,),
        in_specs=[rspec, rspec, rspec, wspec, vspec, vspec, vspec,
                  wspec, vspec, vspec, vspec],
        out_specs=rspec,
    )(x, p0, p1, w1t, b1, g1, be1, w2t, b2, g0, be0)


def _heads(h, p0, p1, w1t, b1, g1, be1, w2t, b2, wct, bc, wst, bs, wmt, bm):
    rspec = pl.BlockSpec((_BM, D), _row)
    wspec = pl.BlockSpec((D, D), _fixed)
    vspec = pl.BlockSpec((1, D), _fixed)
    return pl.pallas_call(
        _head_body,
        out_shape=(
            jax.ShapeDtypeStruct((N, 40), jnp.float32),
            jax.ShapeDtypeStruct((N, 40), jnp.float32),
            jax.ShapeDtypeStruct((N, 2), jnp.float32),
        ),
        grid=(N // _BM,),
        in_specs=[rspec, rspec, rspec, wspec, vspec, vspec, vspec,
                  wspec, vspec,
                  pl.BlockSpec((D, 40), _fixed), pl.BlockSpec((1, 40), _fixed),
                  pl.BlockSpec((D, 40), _fixed), pl.BlockSpec((1, 40), _fixed),
                  pl.BlockSpec((D, 2), _fixed), pl.BlockSpec((1, 2), _fixed)],
        out_specs=(
            pl.BlockSpec((_BM, 40), _row),
            pl.BlockSpec((_BM, 40), _row),
            pl.BlockSpec((_BM, 2), _row),
        ),
    )(h, p0, p1, w1t, b1, g1, be1, w2t, b2, wct, bc, wst, bs, wmt, bm)


def kernel(x, edge_index, params):
    src = edge_index[0].astype(jnp.int32)
    dst = edge_index[1].astype(jnp.int32)
    src3 = src
    dst3 = dst
    zeros = jnp.zeros((N, D), jnp.float32)

    c0, c1 = params["conv0"], params["conv1"]

    def vec(v):
        return v.reshape(1, -1)

    parts0 = _segment_sum_sc(x, src3, dst3, zeros)
    h1 = _mlp0(
        x, parts0[0], parts0[1],
        c0["lin1"]["W"].T, vec(c0["lin1"]["b"]), vec(c0["bn"]["g"]), vec(c0["bn"]["be"]),
        c0["lin2"]["W"].T, vec(c0["lin2"]["b"]),
        vec(params["bn0"]["g"]), vec(params["bn0"]["be"]),
    )

    parts1 = _segment_sum_sc(h1, src3, dst3, zeros)
    wmt = jnp.concatenate([params["homo"]["W"].T, params["ent"]["W"].T], axis=1)
    bm = jnp.concatenate([params["homo"]["b"], params["ent"]["b"]]).reshape(1, 2)
    main, sim, he = _heads(
        h1, parts1[0], parts1[1],
        c1["lin1"]["W"].T, vec(c1["lin1"]["b"]), vec(c1["bn"]["g"]), vec(c1["bn"]["be"]),
        c1["lin2"]["W"].T, vec(c1["lin2"]["b"]),
        params["cls"]["W"].T, vec(params["cls"]["b"]),
        params["sim"]["W"].T, vec(params["sim"]["b"]),
        wmt, bm,
    )
    return main, sim, he[:, 0], he[:, 1]


# speedup vs baseline: 2.4681x; 1.3913x over previous
import functools
import math

import jax
import jax.numpy as jnp
from jax import lax
from jax.experimental import pallas as pl
from jax.experimental.pallas import tpu as pltpu
from jax.experimental.pallas import tpu_sc as plsc

N = 10000
E = 320000
D = 128
NC = 2
NS = 16
NW = NC * NS
EPT = E // NW
CHUNK = 80
NCHUNK = 125
EPAD = NCHUNK * CHUNK - EPT
NBUF = 1
NACC = N
RPS = 624
TAIL = N - NS * RPS

_INV = 1.0 / math.sqrt(1.0 + 1e-5)


def _segment_sum_sc(h, src3, dst3, zeros):
    mesh = plsc.VectorSubcoreMesh(
        core_axis_name="c", subcore_axis_name="s", num_cores=NC, num_subcores=NS
    )

    @functools.partial(
        pl.kernel,
        out_type=jax.ShapeDtypeStruct((NC, N, D), jnp.float32),
        mesh=mesh,
        scratch_types=[
            pltpu.VMEM((128, CHUNK), jnp.int32),
            pltpu.VMEM((128, CHUNK), jnp.int32),
            [pltpu.VMEM((CHUNK, D), jnp.float32) for _ in range(NBUF)],
            pltpu.VMEM_SHARED((NACC, D), jnp.float32),
            [pltpu.SemaphoreType.DMA for _ in range(NBUF)],
            [pltpu.SemaphoreType.DMA for _ in range(NBUF)],
        ],
    )
    def k(h_hbm, src_hbm, dst_hbm, z_hbm, out_hbm, srcv, dstv, rows, acc,
          gsem, ssem):
        cid = lax.axis_index("c")
        sid = lax.axis_index("s")
        wid = sid * NC + cid
        r0 = sid * RPS

        pltpu.sync_copy(src_hbm.at[pl.ds(wid * 128, 128), :], srcv)
        pltpu.sync_copy(dst_hbm.at[pl.ds(wid * 128, 128), :], dstv)
        pltpu.sync_copy(z_hbm.at[pl.ds(r0, RPS)], acc.at[pl.ds(r0, RPS)])

        @pl.when(sid == NS - 1)
        def _():
            pltpu.sync_copy(z_hbm.at[pl.ds(NS * RPS, TAIL)],
                            acc.at[pl.ds(NS * RPS, TAIL)])

        plsc.subcore_barrier()

        @pl.loop(0, NCHUNK)
        def _(i):
            pltpu.async_copy(h_hbm.at[srcv.at[i]], rows[0], gsem[0]).wait()
            pltpu.sync_copy(rows[0], acc.at[dstv.at[i]], add=True)

        plsc.subcore_barrier()
        pltpu.sync_copy(acc.at[pl.ds(r0, RPS)], out_hbm.at[cid].at[pl.ds(r0, RPS)])

        @pl.when(sid == NS - 1)
        def _():
            pltpu.sync_copy(acc.at[pl.ds(NS * RPS, TAIL)],
                            out_hbm.at[cid].at[pl.ds(NS * RPS, TAIL)])

    return k(h, src3, dst3, zeros)


def _mlp0_body(x_ref, p0_ref, p1_ref, w1t_ref, b1_ref, g1_ref, be1_ref,
               w2t_ref, b2_ref, g0_ref, be0_ref, o_ref):
    t = x_ref[...] + p0_ref[...] + p1_ref[...]
    a = jnp.dot(t, w1t_ref[...], preferred_element_type=jnp.float32) + b1_ref[...]
    a = jnp.maximum(a * (_INV * g1_ref[...]) + be1_ref[...], 0.0)
    h = jnp.dot(a, w2t_ref[...], preferred_element_type=jnp.float32) + b2_ref[...]
    o_ref[...] = jnp.maximum(h * (_INV * g0_ref[...]) + be0_ref[...], 0.0)


def _head_body(h_ref, p0_ref, p1_ref, w1t_ref, b1_ref, g1_ref, be1_ref,
               w2t_ref, b2_ref, wct_ref, bc_ref, wst_ref, bs_ref,
               wmt_ref, bm_ref, main_ref, sim_ref, he_ref):
    t = h_ref[...] + p0_ref[...] + p1_ref[...]
    a = jnp.dot(t, w1t_ref[...], preferred_element_type=jnp.float32) + b1_ref[...]
    a = jnp.maximum(a * (_INV * g1_ref[...]) + be1_ref[...], 0.0)
    h2 = jnp.dot(a, w2t_ref[...], preferred_element_type=jnp.float32) + b2_ref[...]

    main = jnp.dot(h2, wct_ref[...], preferred_element_type=jnp.float32) + bc_ref[...]
    m = jnp.max(main, axis=-1, keepdims=True)
    s = main - m
    main_ref[...] = s - jnp.log(jnp.sum(jnp.exp(s), axis=-1, keepdims=True))

    sim = jnp.dot(h2, wst_ref[...], preferred_element_type=jnp.float32) + bs_ref[...]
    ms = jnp.max(sim, axis=-1, keepdims=True)
    es = jnp.exp(sim - ms)
    sim_ref[...] = es / jnp.sum(es, axis=-1, keepdims=True)

    he = jnp.dot(h2, wmt_ref[...], preferred_element_type=jnp.float32) + bm_ref[...]
    he_ref[...] = 1.0 / (1.0 + jnp.exp(-he))


_BM = 1000


def _row(i):
    return (i, 0)


def _fixed(i):
    return (0, 0)


def _mlp0(x, p0, p1, w1t, b1, g1, be1, w2t, b2, g0, be0):
    rspec = pl.BlockSpec((_BM, D), _row)
    wspec = pl.BlockSpec((D, D), _fixed)
    vspec = pl.BlockSpec((1, D), _fixed)
    return pl.pallas_call(
        _mlp0_body,
        out_shape=jax.ShapeDtypeStruct((N, D), jnp.float32),
        grid=(N // _BM,),
        in_specs=[rspec, rspec, rspec, wspec, vspec, vspec, vspec,
                  wspec, vspec, vspec, vspec],
        out_specs=rspec,
    )(x, p0, p1, w1t, b1, g1, be1, w2t, b2, g0, be0)


def _heads(h, p0, p1, w1t, b1, g1, be1, w2t, b2, wct, bc, wst, bs, wmt, bm):
    rspec = pl.BlockSpec((_BM, D), _row)
    wspec = pl.BlockSpec((D, D), _fixed)
    vspec = pl.BlockSpec((1, D), _fixed)
    return pl.pallas_call(
        _head_body,
        out_shape=(
            jax.ShapeDtypeStruct((N, 40), jnp.float32),
            jax.ShapeDtypeStruct((N, 40), jnp.float32),
            jax.ShapeDtypeStruct((N, 2), jnp.float32),
        ),
        grid=(N // _BM,),
        in_specs=[rspec, rspec, rspec, wspec, vspec, vspec, vspec,
                  wspec, vspec,
                  pl.BlockSpec((D, 40), _fixed), pl.BlockSpec((1, 40), _fixed),
                  pl.BlockSpec((D, 40), _fixed), pl.BlockSpec((1, 40), _fixed),
                  pl.BlockSpec((D, 2), _fixed), pl.BlockSpec((1, 2), _fixed)],
        out_specs=(
            pl.BlockSpec((_BM, 40), _row),
            pl.BlockSpec((_BM, 40), _row),
            pl.BlockSpec((_BM, 2), _row),
        ),
    )(h, p0, p1, w1t, b1, g1, be1, w2t, b2, wct, bc, wst, bs, wmt, bm)


def kernel(x, edge_index, params):
    src = edge_index[0].astype(jnp.int32)
    dst = edge_index[1].astype(jnp.int32)
    def chunked(ix):
        m = ix.reshape(NW, NCHUNK, CHUNK)
        return jnp.pad(m, ((0, 0), (0, 128 - NCHUNK), (0, 0))).reshape(
            NW * 128, CHUNK)

    src3 = chunked(src)
    dst3 = chunked(dst)
    zeros = jnp.zeros((N, D), jnp.float32)

    c0, c1 = params["conv0"], params["conv1"]

    def vec(v):
        return v.reshape(1, -1)

    parts0 = _segment_sum_sc(x, src3, dst3, zeros)
    h1 = _mlp0(
        x, parts0[0], parts0[1],
        c0["lin1"]["W"].T, vec(c0["lin1"]["b"]), vec(c0["bn"]["g"]), vec(c0["bn"]["be"]),
        c0["lin2"]["W"].T, vec(c0["lin2"]["b"]),
        vec(params["bn0"]["g"]), vec(params["bn0"]["be"]),
    )

    parts1 = _segment_sum_sc(h1, src3, dst3, zeros)
    wmt = jnp.concatenate([params["homo"]["W"].T, params["ent"]["W"].T], axis=1)
    bm = jnp.concatenate([params["homo"]["b"], params["ent"]["b"]]).reshape(1, 2)
    main, sim, he = _heads(
        h1, parts1[0], parts1[1],
        c1["lin1"]["W"].T, vec(c1["lin1"]["b"]), vec(c1["bn"]["g"]), vec(c1["bn"]["be"]),
        c1["lin2"]["W"].T, vec(c1["lin2"]["b"]),
        params["cls"]["W"].T, vec(params["cls"]["b"]),
        params["sim"]["W"].T, vec(params["sim"]["b"]),
        wmt, bm,
    )
    return main, sim, he[:, 0], he[:, 1]


# speedup vs baseline: 2.8040x; 1.1361x over previous
import functools
import math

import jax
import jax.numpy as jnp
from jax import lax
from jax.experimental import pallas as pl
from jax.experimental.pallas import tpu as pltpu
from jax.experimental.pallas import tpu_sc as plsc

N = 10000
E = 320000
D = 128
NC = 2
NS = 16
NW = NC * NS
EPT = E // NW
CHUNK = 128
NFULL = 78
TAILC = EPT - NFULL * CHUNK
NBUF = 1
NACC = N
RPS = 624
TAIL = N - NS * RPS

_INV = 1.0 / math.sqrt(1.0 + 1e-5)


def _segment_sum_sc(h, src3, dst3, srct, dstt, zeros):
    mesh = plsc.VectorSubcoreMesh(
        core_axis_name="c", subcore_axis_name="s", num_cores=NC, num_subcores=NS
    )

    @functools.partial(
        pl.kernel,
        out_type=jax.ShapeDtypeStruct((NC, N, D), jnp.float32),
        mesh=mesh,
        scratch_types=[
            pltpu.VMEM((80, CHUNK), jnp.int32),
            pltpu.VMEM((80, CHUNK), jnp.int32),
            pltpu.VMEM((TAILC,), jnp.int32),
            pltpu.VMEM((TAILC,), jnp.int32),
            [pltpu.VMEM((CHUNK, D), jnp.float32) for _ in range(NBUF)],
            pltpu.VMEM_SHARED((NACC, D), jnp.float32),
            [pltpu.SemaphoreType.DMA for _ in range(NBUF)],
            [pltpu.SemaphoreType.DMA for _ in range(NBUF)],
        ],
    )
    def k(h_hbm, src_hbm, dst_hbm, srct_hbm, dstt_hbm, z_hbm, out_hbm,
          srcv, dstv, srctv, dsttv, rows, acc, gsem, ssem):
        cid = lax.axis_index("c")
        sid = lax.axis_index("s")
        wid = sid * NC + cid
        r0 = sid * RPS

        pltpu.sync_copy(src_hbm.at[pl.ds(wid * 80, 80), :], srcv)
        pltpu.sync_copy(dst_hbm.at[pl.ds(wid * 80, 80), :], dstv)
        pltpu.sync_copy(srct_hbm.at[pl.ds(wid * TAILC, TAILC)], srctv)
        pltpu.sync_copy(dstt_hbm.at[pl.ds(wid * TAILC, TAILC)], dsttv)
        pltpu.sync_copy(z_hbm.at[pl.ds(r0, RPS)], acc.at[pl.ds(r0, RPS)])

        @pl.when(sid == NS - 1)
        def _():
            pltpu.sync_copy(z_hbm.at[pl.ds(NS * RPS, TAIL)],
                            acc.at[pl.ds(NS * RPS, TAIL)])

        plsc.subcore_barrier()

        @pl.loop(0, NFULL)
        def _(i):
            pltpu.async_copy(h_hbm.at[srcv.at[i]], rows[0], gsem[0]).wait()
            pltpu.sync_copy(rows[0], acc.at[dstv.at[i]], add=True)

        pltpu.async_copy(h_hbm.at[srctv], rows[0].at[pl.ds(0, TAILC), :],
                         gsem[0]).wait()
        pltpu.sync_copy(rows[0].at[pl.ds(0, TAILC), :], acc.at[dsttv],
                        add=True)

        plsc.subcore_barrier()
        pltpu.sync_copy(acc.at[pl.ds(r0, RPS)], out_hbm.at[cid].at[pl.ds(r0, RPS)])

        @pl.when(sid == NS - 1)
        def _():
            pltpu.sync_copy(acc.at[pl.ds(NS * RPS, TAIL)],
                            out_hbm.at[cid].at[pl.ds(NS * RPS, TAIL)])

    return k(h, src3, dst3, srct, dstt, zeros)


def _mlp0_body(x_ref, p0_ref, p1_ref, w1t_ref, b1_ref, g1_ref, be1_ref,
               w2t_ref, b2_ref, g0_ref, be0_ref, o_ref):
    t = x_ref[...] + p0_ref[...] + p1_ref[...]
    a = jnp.dot(t, w1t_ref[...], preferred_element_type=jnp.float32) + b1_ref[...]
    a = jnp.maximum(a * (_INV * g1_ref[...]) + be1_ref[...], 0.0)
    h = jnp.dot(a, w2t_ref[...], preferred_element_type=jnp.float32) + b2_ref[...]
    o_ref[...] = jnp.maximum(h * (_INV * g0_ref[...]) + be0_ref[...], 0.0)


def _head_body(h_ref, p0_ref, p1_ref, w1t_ref, b1_ref, g1_ref, be1_ref,
               w2t_ref, b2_ref, wct_ref, bc_ref, wst_ref, bs_ref,
               wmt_ref, bm_ref, main_ref, sim_ref, he_ref):
    t = h_ref[...] + p0_ref[...] + p1_ref[...]
    a = jnp.dot(t, w1t_ref[...], preferred_element_type=jnp.float32) + b1_ref[...]
    a = jnp.maximum(a * (_INV * g1_ref[...]) + be1_ref[...], 0.0)
    h2 = jnp.dot(a, w2t_ref[...], preferred_element_type=jnp.float32) + b2_ref[...]

    main = jnp.dot(h2, wct_ref[...], preferred_element_type=jnp.float32) + bc_ref[...]
    m = jnp.max(main, axis=-1, keepdims=True)
    s = main - m
    main_ref[...] = s - jnp.log(jnp.sum(jnp.exp(s), axis=-1, keepdims=True))

    sim = jnp.dot(h2, wst_ref[...], preferred_element_type=jnp.float32) + bs_ref[...]
    ms = jnp.max(sim, axis=-1, keepdims=True)
    es = jnp.exp(sim - ms)
    sim_ref[...] = es / jnp.sum(es, axis=-1, keepdims=True)

    he = jnp.dot(h2, wmt_ref[...], preferred_element_type=jnp.float32) + bm_ref[...]
    he_ref[...] = 1.0 / (1.0 + jnp.exp(-he))


_BM = 1000


def _row(i):
    return (i, 0)


def _fixed(i):
    return (0, 0)


def _mlp0(x, p0, p1, w1t, b1, g1, be1, w2t, b2, g0, be0):
    rspec = pl.BlockSpec((_BM, D), _row)
    wspec = pl.BlockSpec((D, D), _fixed)
    vspec = pl.BlockSpec((1, D), _fixed)
    return pl.pallas_call(
        _mlp0_body,
        out_shape=jax.ShapeDtypeStruct((N, D), jnp.float32),
        grid=(N // _BM,),
        in_specs=[rspec, rspec, rspec, wspec, vspec, vspec, vspec,
                  wspec, vspec, vspec, vspec],
        out_specs=rspec,
    )(x, p0, p1, w1t, b1, g1, be1, w2t, b2, g0, be0)


def _heads(h, p0, p1, w1t, b1, g1, be1, w2t, b2, wct, bc, wst, bs, wmt, bm):
    rspec = pl.BlockSpec((_BM, D), _row)
    wspec = pl.BlockSpec((D, D), _fixed)
    vspec = pl.BlockSpec((1, D), _fixed)
    return pl.pallas_call(
        _head_body,
        out_shape=(
            jax.ShapeDtypeStruct((N, 40), jnp.float32),
            jax.ShapeDtypeStruct((N, 40), jnp.float32),
            jax.ShapeDtypeStruct((N, 2), jnp.float32),
        ),
        grid=(N // _BM,),
        in_specs=[rspec, rspec, rspec, wspec, vspec, vspec, vspec,
                  wspec, vspec,
                  pl.BlockSpec((D, 40), _fixed), pl.BlockSpec((1, 40), _fixed),
                  pl.BlockSpec((D, 40), _fixed), pl.BlockSpec((1, 40), _fixed),
                  pl.BlockSpec((D, 2), _fixed), pl.BlockSpec((1, 2), _fixed)],
        out_specs=(
            pl.BlockSpec((_BM, 40), _row),
            pl.BlockSpec((_BM, 40), _row),
            pl.BlockSpec((_BM, 2), _row),
        ),
    )(h, p0, p1, w1t, b1, g1, be1, w2t, b2, wct, bc, wst, bs, wmt, bm)


def kernel(x, edge_index, params):
    src = edge_index[0].astype(jnp.int32)
    dst = edge_index[1].astype(jnp.int32)
    def chunked(ix):
        m = ix.reshape(NW, EPT)
        full = m[:, :NFULL * CHUNK].reshape(NW, NFULL, CHUNK)
        full = jnp.pad(full, ((0, 0), (0, 80 - NFULL), (0, 0))).reshape(
            NW * 80, CHUNK)
        t = m[:, NFULL * CHUNK:].reshape(NW * TAILC)
        return full, t

    src3, srct = chunked(src)
    dst3, dstt = chunked(dst)
    zeros = jnp.zeros((N, D), jnp.float32)

    c0, c1 = params["conv0"], params["conv1"]

    def vec(v):
        return v.reshape(1, -1)

    parts0 = _segment_sum_sc(x, src3, dst3, srct, dstt, zeros)
    h1 = _mlp0(
        x, parts0[0], parts0[1],
        c0["lin1"]["W"].T, vec(c0["lin1"]["b"]), vec(c0["bn"]["g"]), vec(c0["bn"]["be"]),
        c0["lin2"]["W"].T, vec(c0["lin2"]["b"]),
        vec(params["bn0"]["g"]), vec(params["bn0"]["be"]),
    )

    parts1 = _segment_sum_sc(h1, src3, dst3, srct, dstt, zeros)
    wmt = jnp.concatenate([params["homo"]["W"].T, params["ent"]["W"].T], axis=1)
    bm = jnp.concatenate([params["homo"]["b"], params["ent"]["b"]]).reshape(1, 2)
    main, sim, he = _heads(
        h1, parts1[0], parts1[1],
        c1["lin1"]["W"].T, vec(c1["lin1"]["b"]), vec(c1["bn"]["g"]), vec(c1["bn"]["be"]),
        c1["lin2"]["W"].T, vec(c1["lin2"]["b"]),
        params["cls"]["W"].T, vec(params["cls"]["b"]),
        params["sim"]["W"].T, vec(params["sim"]["b"]),
        wmt, bm,
    )
    return main, sim, he[:, 0], he[:, 1]


# speedup vs baseline: 3.4136x; 1.2174x over previous
import functools
import math

import jax
import jax.numpy as jnp
from jax import lax
from jax.experimental import pallas as pl
from jax.experimental.pallas import tpu as pltpu
from jax.experimental.pallas import tpu_sc as plsc

N = 10000
E = 320000
D = 128
DH = D // 2
NC = 2
NS = 16
EPS = E // NS
CHUNK = 128
NFULL = EPS // CHUNK
TAILC = EPS - NFULL * CHUNK
IDXROWS = 160
NBUF = 4
NACC = N + 16
RPS = 624
ZTAIL = NACC - NS * RPS
WTAIL = N - NS * RPS

_INV = 1.0 / math.sqrt(1.0 + 1e-5)


def _segment_sum_sc(hL, hR, srcm, srct, dstm, dstt, zeros):
    mesh = plsc.VectorSubcoreMesh(
        core_axis_name="c", subcore_axis_name="s", num_cores=NC, num_subcores=NS
    )

    @functools.partial(
        pl.kernel,
        out_type=jax.ShapeDtypeStruct((NC, N, DH), jnp.float32),
        mesh=mesh,
        compiler_params=pltpu.CompilerParams(use_tc_tiling_on_sc=False),
        scratch_types=[
            pltpu.VMEM((IDXROWS, CHUNK), jnp.int32),
            pltpu.VMEM((IDXROWS, CHUNK), jnp.int32),
            pltpu.VMEM((TAILC,), jnp.int32),
            pltpu.VMEM((TAILC,), jnp.int32),
            [pltpu.VMEM((CHUNK, DH), jnp.float32) for _ in range(NBUF)],
            pltpu.VMEM((TAILC, DH), jnp.float32),
            pltpu.VMEM_SHARED((NACC, DH), jnp.float32),
            [pltpu.SemaphoreType.DMA for _ in range(NBUF)],
            [pltpu.SemaphoreType.DMA for _ in range(NBUF)],
        ],
    )
    def k(hL_hbm, hR_hbm, src_hbm, srct_hbm, dst_hbm, dstt_hbm, z_hbm,
          out_hbm, srcv, dstv, srctv, dsttv, rows, rowst, acc, gsem, ssem):
        cid = lax.axis_index("c")
        sid = lax.axis_index("s")
        r0 = sid * RPS

        pltpu.sync_copy(src_hbm.at[pl.ds(sid * IDXROWS, IDXROWS), :], srcv)
        pltpu.sync_copy(dst_hbm.at[pl.ds(sid * IDXROWS, IDXROWS), :], dstv)
        pltpu.sync_copy(srct_hbm.at[pl.ds(sid * TAILC, TAILC)], srctv)
        pltpu.sync_copy(dstt_hbm.at[pl.ds(sid * TAILC, TAILC)], dsttv)
        pltpu.sync_copy(z_hbm.at[pl.ds(r0, RPS)], acc.at[pl.ds(r0, RPS)])

        @pl.when(sid == NS - 1)
        def _():
            pltpu.sync_copy(z_hbm.at[pl.ds(NS * RPS, ZTAIL)],
                            acc.at[pl.ds(NS * RPS, ZTAIL)])

        plsc.subcore_barrier()

        def run_pass(h_hbm):
            def gather(i, b):
                return pltpu.make_async_copy(h_hbm.at[srcv.at[i]], rows[b],
                                             gsem[b])

            def scatter(i, b):
                return pltpu.make_async_copy(rows[b], acc.at[dstv.at[i]],
                                             ssem[b])

            for b in range(NBUF):
                gather(b, b).start()

            @pl.loop(0, (NFULL - NBUF) // NBUF)
            def _(j):
                i0 = j * NBUF
                for b in range(NBUF):
                    gather(i0 + b, b).wait()
                    scatter(i0 + b, b).start(add=True)
                for b in range(NBUF):
                    scatter(i0 + b, b).wait()
                    gather(i0 + NBUF + b, b).start()

            i0 = NFULL - NBUF
            for b in range(NBUF):
                gather(i0 + b, b).wait()
                scatter(i0 + b, b).start(add=True)
            for b in range(NBUF):
                scatter(i0 + b, b).wait()

            pltpu.async_copy(h_hbm.at[srctv], rowst, gsem[0]).wait()
            pltpu.sync_copy(rowst, acc.at[dsttv], add=True)

        @pl.when(cid == 0)
        def _():
            run_pass(hL_hbm)

        @pl.when(cid == 1)
        def _():
            run_pass(hR_hbm)

        plsc.subcore_barrier()
        pltpu.sync_copy(acc.at[pl.ds(r0, RPS)], out_hbm.at[cid].at[pl.ds(r0, RPS)])

        @pl.when(sid == NS - 1)
        def _():
            pltpu.sync_copy(acc.at[pl.ds(NS * RPS, WTAIL)],
                            out_hbm.at[cid].at[pl.ds(NS * RPS, WTAIL)])

    return k(hL, hR, srcm, srct, dstm, dstt, zeros)


def _mlp0_body(x_ref, pL_ref, pR_ref, w1t_ref, b1_ref, g1_ref, be1_ref,
               w2t_ref, b2_ref, g0_ref, be0_ref, o_ref):
    t = x_ref[...] + jnp.concatenate([pL_ref[...], pR_ref[...]], axis=1)
    a = jnp.dot(t, w1t_ref[...], preferred_element_type=jnp.float32) + b1_ref[...]
    a = jnp.maximum(a * (_INV * g1_ref[...]) + be1_ref[...], 0.0)
    h = jnp.dot(a, w2t_ref[...], preferred_element_type=jnp.float32) + b2_ref[...]
    o_ref[...] = jnp.maximum(h * (_INV * g0_ref[...]) + be0_ref[...], 0.0)


def _head_body(h_ref, pL_ref, pR_ref, w1t_ref, b1_ref, g1_ref, be1_ref,
               w2t_ref, b2_ref, wct_ref, bc_ref, wst_ref, bs_ref,
               wmt_ref, bm_ref, main_ref, sim_ref, he_ref):
    t = h_ref[...] + jnp.concatenate([pL_ref[...], pR_ref[...]], axis=1)
    a = jnp.dot(t, w1t_ref[...], preferred_element_type=jnp.float32) + b1_ref[...]
    a = jnp.maximum(a * (_INV * g1_ref[...]) + be1_ref[...], 0.0)
    h2 = jnp.dot(a, w2t_ref[...], preferred_element_type=jnp.float32) + b2_ref[...]

    main = jnp.dot(h2, wct_ref[...], preferred_element_type=jnp.float32) + bc_ref[...]
    m = jnp.max(main, axis=-1, keepdims=True)
    s = main - m
    main_ref[...] = s - jnp.log(jnp.sum(jnp.exp(s), axis=-1, keepdims=True))

    sim = jnp.dot(h2, wst_ref[...], preferred_element_type=jnp.float32) + bs_ref[...]
    ms = jnp.max(sim, axis=-1, keepdims=True)
    es = jnp.exp(sim - ms)
    sim_ref[...] = es / jnp.sum(es, axis=-1, keepdims=True)

    he = jnp.dot(h2, wmt_ref[...], preferred_element_type=jnp.float32) + bm_ref[...]
    he_ref[...] = 1.0 / (1.0 + jnp.exp(-he))


_BM = 1000


def _row(i):
    return (i, 0)


def _fixed(i):
    return (0, 0)


def _mlp0(x, pL, pR, w1t, b1, g1, be1, w2t, b2, g0, be0):
    rspec = pl.BlockSpec((_BM, D), _row)
    pspec = pl.BlockSpec((_BM, DH), _row)
    wspec = pl.BlockSpec((D, D), _fixed)
    vspec = pl.BlockSpec((1, D), _fixed)
    return pl.pallas_call(
        _mlp0_body,
        out_shape=jax.ShapeDtypeStruct((N, D), jnp.float32),
        grid=(N // _BM,),
        in_specs=[rspec, pspec, pspec, wspec, vspec, vspec, vspec,
                  wspec, vspec, vspec, vspec],
        out_specs=rspec,
    )(x, pL, pR, w1t, b1, g1, be1, w2t, b2, g0, be0)


def _heads(h, pL, pR, w1t, b1, g1, be1, w2t, b2, wct, bc, wst, bs, wmt, bm):
    rspec = pl.BlockSpec((_BM, D), _row)
    pspec = pl.BlockSpec((_BM, DH), _row)
    wspec = pl.BlockSpec((D, D), _fixed)
    vspec = pl.BlockSpec((1, D), _fixed)
    return pl.pallas_call(
        _head_body,
        out_shape=(
            jax.ShapeDtypeStruct((N, 40), jnp.float32),
            jax.ShapeDtypeStruct((N, 40), jnp.float32),
            jax.ShapeDtypeStruct((N, 2), jnp.float32),
        ),
        grid=(N // _BM,),
        in_specs=[rspec, pspec, pspec, wspec, vspec, vspec, vspec,
                  wspec, vspec,
                  pl.BlockSpec((D, 40), _fixed), pl.BlockSpec((1, 40), _fixed),
                  pl.BlockSpec((D, 40), _fixed), pl.BlockSpec((1, 40), _fixed),
                  pl.BlockSpec((D, 2), _fixed), pl.BlockSpec((1, 2), _fixed)],
        out_specs=(
            pl.BlockSpec((_BM, 40), _row),
            pl.BlockSpec((_BM, 40), _row),
            pl.BlockSpec((_BM, 2), _row),
        ),
    )(h, pL, pR, w1t, b1, g1, be1, w2t, b2, wct, bc, wst, bs, wmt, bm)


def kernel(x, edge_index, params):
    src = edge_index[0].astype(jnp.int32)
    dst = edge_index[1].astype(jnp.int32)

    def chunked(ix):
        m = ix.reshape(NS, EPS)
        full = m[:, :NFULL * CHUNK].reshape(NS, NFULL, CHUNK)
        full = jnp.pad(full, ((0, 0), (0, IDXROWS - NFULL), (0, 0))).reshape(
            NS * IDXROWS, CHUNK)
        t = m[:, NFULL * CHUNK:].reshape(NS * TAILC)
        return full, t

    srcm, srct = chunked(src)
    dstm, dstt = chunked(dst)
    zeros = jnp.zeros((NACC, DH), jnp.float32)

    c0, c1 = params["conv0"], params["conv1"]

    def vec(v):
        return v.reshape(1, -1)

    parts0 = _segment_sum_sc(x[:, :DH], x[:, DH:], srcm, srct, dstm, dstt,
                             zeros)
    h1 = _mlp0(
        x, parts0[0], parts0[1],
        c0["lin1"]["W"].T, vec(c0["lin1"]["b"]), vec(c0["bn"]["g"]), vec(c0["bn"]["be"]),
        c0["lin2"]["W"].T, vec(c0["lin2"]["b"]),
        vec(params["bn0"]["g"]), vec(params["bn0"]["be"]),
    )

    parts1 = _segment_sum_sc(h1[:, :DH], h1[:, DH:], srcm, srct, dstm, dstt,
                             zeros)
    wmt = jnp.concatenate([params["homo"]["W"].T, params["ent"]["W"].T], axis=1)
    bm = jnp.concatenate([params["homo"]["b"], params["ent"]["b"]]).reshape(1, 2)
    main, sim, he = _heads(
        h1, parts1[0], parts1[1],
        c1["lin1"]["W"].T, vec(c1["lin1"]["b"]), vec(c1["bn"]["g"]), vec(c1["bn"]["be"]),
        c1["lin2"]["W"].T, vec(c1["lin2"]["b"]),
        params["cls"]["W"].T, vec(params["cls"]["b"]),
        params["sim"]["W"].T, vec(params["sim"]["b"]),
        wmt, bm,
    )
    return main, sim, he[:, 0], he[:, 1]


# speedup vs baseline: 3.4929x; 1.0232x over previous
import functools
import math

import jax
import jax.numpy as jnp
from jax import lax
from jax.experimental import pallas as pl
from jax.experimental.pallas import tpu as pltpu
from jax.experimental.pallas import tpu_sc as plsc

N = 10000
E = 320000
D = 128
DH = D // 2
NC = 2
NS = 16
EPS = E // NS
CHUNK = 128
NFULL = EPS // CHUNK
TAILC = EPS - NFULL * CHUNK
IDXROWS = 160
NBUF = 4
NACC = N
RPS = 624
WTAIL = N - NS * RPS

_INV = 1.0 / math.sqrt(1.0 + 1e-5)


def _segment_sum_sc(hL, hR, srcm, srct, dstm, dstt):
    mesh = plsc.VectorSubcoreMesh(
        core_axis_name="c", subcore_axis_name="s", num_cores=NC, num_subcores=NS
    )

    @functools.partial(
        pl.kernel,
        out_type=jax.ShapeDtypeStruct((NC, N, DH), jnp.float32),
        mesh=mesh,
        compiler_params=pltpu.CompilerParams(use_tc_tiling_on_sc=False),
        scratch_types=[
            pltpu.VMEM((IDXROWS, CHUNK), jnp.int32),
            pltpu.VMEM((IDXROWS, CHUNK), jnp.int32),
            pltpu.VMEM((TAILC,), jnp.int32),
            pltpu.VMEM((TAILC,), jnp.int32),
            [pltpu.VMEM((CHUNK, DH), jnp.float32) for _ in range(NBUF)],
            pltpu.VMEM((TAILC, DH), jnp.float32),
            pltpu.VMEM_SHARED((NACC, DH), jnp.float32),
            [pltpu.SemaphoreType.DMA for _ in range(NBUF)],
            [pltpu.SemaphoreType.DMA for _ in range(NBUF)],
        ],
    )
    def k(hL_hbm, hR_hbm, src_hbm, srct_hbm, dst_hbm, dstt_hbm,
          out_hbm, srcv, dstv, srctv, dsttv, rows, rowst, acc, gsem, ssem):
        cid = lax.axis_index("c")
        sid = lax.axis_index("s")
        r0 = sid * RPS

        pltpu.sync_copy(src_hbm.at[pl.ds(sid * IDXROWS, IDXROWS), :], srcv)
        pltpu.sync_copy(dst_hbm.at[pl.ds(sid * IDXROWS, IDXROWS), :], dstv)
        pltpu.sync_copy(srct_hbm.at[pl.ds(sid * TAILC, TAILC)], srctv)
        pltpu.sync_copy(dstt_hbm.at[pl.ds(sid * TAILC, TAILC)], dsttv)
        @pl.when(cid == 0)
        def _():
            pltpu.sync_copy(hL_hbm.at[pl.ds(r0, RPS)], acc.at[pl.ds(r0, RPS)])

            @pl.when(sid == NS - 1)
            def _():
                pltpu.sync_copy(hL_hbm.at[pl.ds(NS * RPS, WTAIL)],
                                acc.at[pl.ds(NS * RPS, WTAIL)])

        @pl.when(cid == 1)
        def _():
            pltpu.sync_copy(hR_hbm.at[pl.ds(r0, RPS)], acc.at[pl.ds(r0, RPS)])

            @pl.when(sid == NS - 1)
            def _():
                pltpu.sync_copy(hR_hbm.at[pl.ds(NS * RPS, WTAIL)],
                                acc.at[pl.ds(NS * RPS, WTAIL)])

        plsc.subcore_barrier()

        def run_pass(h_hbm):
            def gather(i, b):
                return pltpu.make_async_copy(h_hbm.at[srcv.at[i]], rows[b],
                                             gsem[b])

            def scatter(i, b):
                return pltpu.make_async_copy(rows[b], acc.at[dstv.at[i]],
                                             ssem[b])

            for b in range(NBUF):
                gather(b, b).start()

            @pl.loop(0, (NFULL - NBUF) // NBUF)
            def _(j):
                i0 = j * NBUF
                for b in range(NBUF):
                    gather(i0 + b, b).wait()
                    scatter(i0 + b, b).start(add=True)
                for b in range(NBUF):
                    scatter(i0 + b, b).wait()
                    gather(i0 + NBUF + b, b).start()

            i0 = NFULL - NBUF
            for b in range(NBUF):
                gather(i0 + b, b).wait()
                scatter(i0 + b, b).start(add=True)
            for b in range(NBUF):
                scatter(i0 + b, b).wait()

            pltpu.async_copy(h_hbm.at[srctv], rowst, gsem[0]).wait()
            pltpu.sync_copy(rowst, acc.at[dsttv], add=True)

        @pl.when(cid == 0)
        def _():
            run_pass(hL_hbm)

        @pl.when(cid == 1)
        def _():
            run_pass(hR_hbm)

        plsc.subcore_barrier()
        pltpu.sync_copy(acc.at[pl.ds(r0, RPS)], out_hbm.at[cid].at[pl.ds(r0, RPS)])

        @pl.when(sid == NS - 1)
        def _():
            pltpu.sync_copy(acc.at[pl.ds(NS * RPS, WTAIL)],
                            out_hbm.at[cid].at[pl.ds(NS * RPS, WTAIL)])

    return k(hL, hR, srcm, srct, dstm, dstt)


def _mlp0_body(pL_ref, pR_ref, w1t_ref, b1_ref, g1_ref, be1_ref,
               w2t_ref, b2_ref, g0_ref, be0_ref, oL_ref, oR_ref):
    t = jnp.concatenate([pL_ref[...], pR_ref[...]], axis=1)
    a = jnp.dot(t, w1t_ref[...], preferred_element_type=jnp.float32) + b1_ref[...]
    a = jnp.maximum(a * (_INV * g1_ref[...]) + be1_ref[...], 0.0)
    h = jnp.dot(a, w2t_ref[...], preferred_element_type=jnp.float32) + b2_ref[...]
    h = jnp.maximum(h * (_INV * g0_ref[...]) + be0_ref[...], 0.0)
    oL_ref[...] = h[:, :DH]
    oR_ref[...] = h[:, DH:]


def _head_body(pL_ref, pR_ref, w1t_ref, b1_ref, g1_ref, be1_ref,
               w2t_ref, b2_ref, wct_ref, bc_ref, wst_ref, bs_ref,
               wmt_ref, bm_ref, main_ref, sim_ref, he_ref):
    t = jnp.concatenate([pL_ref[...], pR_ref[...]], axis=1)
    a = jnp.dot(t, w1t_ref[...], preferred_element_type=jnp.float32) + b1_ref[...]
    a = jnp.maximum(a * (_INV * g1_ref[...]) + be1_ref[...], 0.0)
    h2 = jnp.dot(a, w2t_ref[...], preferred_element_type=jnp.float32) + b2_ref[...]

    main = jnp.dot(h2, wct_ref[...], preferred_element_type=jnp.float32) + bc_ref[...]
    m = jnp.max(main, axis=-1, keepdims=True)
    s = main - m
    main_ref[...] = s - jnp.log(jnp.sum(jnp.exp(s), axis=-1, keepdims=True))

    sim = jnp.dot(h2, wst_ref[...], preferred_element_type=jnp.float32) + bs_ref[...]
    ms = jnp.max(sim, axis=-1, keepdims=True)
    es = jnp.exp(sim - ms)
    sim_ref[...] = es / jnp.sum(es, axis=-1, keepdims=True)

    he = jnp.dot(h2, wmt_ref[...], preferred_element_type=jnp.float32) + bm_ref[...]
    he_ref[...] = 1.0 / (1.0 + jnp.exp(-he))


_BM = 1000


def _row(i):
    return (i, 0)


def _fixed(i):
    return (0, 0)


def _mlp0(pL, pR, w1t, b1, g1, be1, w2t, b2, g0, be0):
    pspec = pl.BlockSpec((_BM, DH), _row)
    wspec = pl.BlockSpec((D, D), _fixed)
    vspec = pl.BlockSpec((1, D), _fixed)
    return pl.pallas_call(
        _mlp0_body,
        out_shape=(
            jax.ShapeDtypeStruct((N, DH), jnp.float32),
            jax.ShapeDtypeStruct((N, DH), jnp.float32),
        ),
        grid=(N // _BM,),
        in_specs=[pspec, pspec, wspec, vspec, vspec, vspec,
                  wspec, vspec, vspec, vspec],
        out_specs=(pspec, pspec),
    )(pL, pR, w1t, b1, g1, be1, w2t, b2, g0, be0)


def _heads(pL, pR, w1t, b1, g1, be1, w2t, b2, wct, bc, wst, bs, wmt, bm):
    pspec = pl.BlockSpec((_BM, DH), _row)
    wspec = pl.BlockSpec((D, D), _fixed)
    vspec = pl.BlockSpec((1, D), _fixed)
    return pl.pallas_call(
        _head_body,
        out_shape=(
            jax.ShapeDtypeStruct((N, 40), jnp.float32),
            jax.ShapeDtypeStruct((N, 40), jnp.float32),
            jax.ShapeDtypeStruct((N, 2), jnp.float32),
        ),
        grid=(N // _BM,),
        in_specs=[pspec, pspec, wspec, vspec, vspec, vspec,
                  wspec, vspec,
                  pl.BlockSpec((D, 40), _fixed), pl.BlockSpec((1, 40), _fixed),
                  pl.BlockSpec((D, 40), _fixed), pl.BlockSpec((1, 40), _fixed),
                  pl.BlockSpec((D, 2), _fixed), pl.BlockSpec((1, 2), _fixed)],
        out_specs=(
            pl.BlockSpec((_BM, 40), _row),
            pl.BlockSpec((_BM, 40), _row),
            pl.BlockSpec((_BM, 2), _row),
        ),
    )(pL, pR, w1t, b1, g1, be1, w2t, b2, wct, bc, wst, bs, wmt, bm)


def kernel(x, edge_index, params):
    src = edge_index[0].astype(jnp.int32)
    dst = edge_index[1].astype(jnp.int32)

    def chunked(ix):
        m = ix.reshape(NS, EPS)
        full = m[:, :NFULL * CHUNK].reshape(NS, NFULL, CHUNK)
        full = jnp.pad(full, ((0, 0), (0, IDXROWS - NFULL), (0, 0))).reshape(
            NS * IDXROWS, CHUNK)
        t = m[:, NFULL * CHUNK:].reshape(NS * TAILC)
        return full, t

    srcm, srct = chunked(src)
    dstm, dstt = chunked(dst)

    c0, c1 = params["conv0"], params["conv1"]

    def vec(v):
        return v.reshape(1, -1)

    parts0 = _segment_sum_sc(x[:, :DH], x[:, DH:], srcm, srct, dstm, dstt)
    h1L, h1R = _mlp0(
        parts0[0], parts0[1],
        c0["lin1"]["W"].T, vec(c0["lin1"]["b"]), vec(c0["bn"]["g"]), vec(c0["bn"]["be"]),
        c0["lin2"]["W"].T, vec(c0["lin2"]["b"]),
        vec(params["bn0"]["g"]), vec(params["bn0"]["be"]),
    )

    parts1 = _segment_sum_sc(h1L, h1R, srcm, srct, dstm, dstt)
    wmt = jnp.concatenate([params["homo"]["W"].T, params["ent"]["W"].T], axis=1)
    bm = jnp.concatenate([params["homo"]["b"], params["ent"]["b"]]).reshape(1, 2)
    main, sim, he = _heads(
        parts1[0], parts1[1],
        c1["lin1"]["W"].T, vec(c1["lin1"]["b"]), vec(c1["bn"]["g"]), vec(c1["bn"]["be"]),
        c1["lin2"]["W"].T, vec(c1["lin2"]["b"]),
        params["cls"]["W"].T, vec(params["cls"]["b"]),
        params["sim"]["W"].T, vec(params["sim"]["b"]),
        wmt, bm,
    )
    return main, sim, he[:, 0], he[:, 1]
